# baseline (speedup 1.0000x reference)
import functools

import jax
import jax.numpy as jnp
from jax import lax
from jax.experimental import pallas as pl
from jax.experimental.pallas import tpu as pltpu
from jax.experimental.pallas import tpu_sc as plsc

_LANES = 16
_CHUNK = 64


@functools.partial(jax.jit, static_argnums=(2, 3, 4))
def _sc_expand(input_len, table2, B, T, D):
    NC = 2
    NS = 16
    NW = NC * NS
    R = (B * T) // NW
    C = _CHUNK
    n_chunks = R // C
    w_per_b = NW // B

    mesh = plsc.VectorSubcoreMesh(core_axis_name="c", subcore_axis_name="s")

    @functools.partial(
        pl.kernel,
        mesh=mesh,
        out_type=jax.ShapeDtypeStruct((B * T, D), jnp.float32),
        scratch_types=[
            pltpu.VMEM((_LANES,), jnp.int32),
            pltpu.VMEM((C, D), jnp.float32),
            pltpu.SemaphoreType.DMA,
            pltpu.SemaphoreType.DMA,
        ],
    )
    def _k(len_hbm, tab_hbm, out_hbm, lens_v, buf, semC, semZ):
        c = lax.axis_index("c")
        s = lax.axis_index("s")
        wid = s * NC + c
        b = wid // w_per_b
        base_t = (wid % w_per_b) * R
        o_base = b * T + base_t

        pltpu.sync_copy(len_hbm, lens_v.at[pl.ds(0, B)])
        lens16 = lens_v[...]
        len_b = lens16[0]
        for bb in range(1, B):
            len_b = jnp.where(b == bb, lens16[bb], len_b)

        v = jnp.clip(len_b - base_t, 0, R)
        nfull = v // C
        m_rem = v - nfull * C
        pad0 = nfull + jnp.where(m_rem > 0, 1, 0)

        def full_body(j, carry):
            t0 = base_t + j * C
            pltpu.sync_copy(tab_hbm.at[pl.ds(t0, C)], buf)
            pltpu.sync_copy(buf, out_hbm.at[pl.ds(b * T + t0, C)])
            return carry

        lax.fori_loop(0, nfull, full_body, 0)

        zero16 = jnp.zeros((_LANES,), jnp.float32)

        @pl.when(m_rem > 0)
        def _boundary():
            t0 = base_t + nfull * C
            pltpu.sync_copy(tab_hbm.at[pl.ds(t0, C)], buf)

            def zrow(rp, carry):
                for g in range(D // _LANES):
                    buf[rp, pl.ds(g * _LANES, _LANES)] = zero16
                return carry

            lax.fori_loop(m_rem, C, zrow, 0)
            pltpu.sync_copy(buf, out_hbm.at[pl.ds(b * T + t0, C)])

        @pl.when(pad0 < n_chunks)
        def _pads():
            def zrow(rp, carry):
                for g in range(D // _LANES):
                    buf[rp, pl.ds(g * _LANES, _LANES)] = zero16
                return carry

            lax.fori_loop(0, jnp.where(m_rem > 0, m_rem, C), zrow, 0)

            def fire_pad(j, carry):
                t0 = base_t + j * C
                pltpu.make_async_copy(
                    buf, out_hbm.at[pl.ds(b * T + t0, C)], semZ).start()
                return carry

            lax.fori_loop(pad0, n_chunks, fire_pad, 0)

            def drain_pad(j, carry):
                pltpu.make_async_copy(
                    buf, out_hbm.at[pl.ds(o_base, C)], semZ).wait()
                return carry

            lax.fori_loop(pad0, n_chunks, drain_pad, 0)

    return _k(input_len, table2)


def _shift_body(a_ref, b_ref, o_ref):
    o_ref[...] = jnp.concatenate([a_ref[1:], b_ref[:1]], axis=0)


@jax.jit
def _shift_table(pos_table):
    V, D = pos_table.shape
    T = V - 1
    CB = 256
    return pl.pallas_call(
        _shift_body,
        grid=(T // CB,),
        in_specs=[
            pl.BlockSpec((CB, D), lambda r: (r, 0)),
            pl.BlockSpec((CB, D), lambda r: (r + 1, 0)),
        ],
        out_specs=pl.BlockSpec((CB, D), lambda r: (r, 0)),
        out_shape=jax.ShapeDtypeStruct((T, D), jnp.float32),
    )(pos_table, pos_table)


def kernel(input_len, max_len, pos_table):
    del max_len
    V, D = pos_table.shape
    T = V - 1
    B = input_len.shape[0]
    table2 = _shift_table(pos_table)
    out = _sc_expand(input_len, table2, B, T, D)
    return out.reshape(B, T, D)


# speedup vs baseline: 1.0956x; 1.0956x over previous
import functools

import jax
import jax.numpy as jnp
from jax import lax
from jax.experimental import pallas as pl
from jax.experimental.pallas import tpu as pltpu
from jax.experimental.pallas import tpu_sc as plsc

_LANES = 16
_CHUNK = 64
_ZROWS = 56


@functools.partial(jax.jit, static_argnums=(2, 3, 4))
def _sc_expand(input_len, table2, B, T, D):
    NC = 2
    NS = 16
    NW = NC * NS
    C = _CHUNK
    G = (B * T) // C
    gpb = T // C
    my_chunks = G // NW

    mesh = plsc.VectorSubcoreMesh(core_axis_name="c", subcore_axis_name="s")

    @functools.partial(
        pl.kernel,
        mesh=mesh,
        out_type=jax.ShapeDtypeStruct((B * T, D), jnp.float32),
        scratch_types=[
            pltpu.VMEM((_LANES,), jnp.int32),
            pltpu.VMEM((_ZROWS, D), jnp.float32),
            pltpu.VMEM((C, D), jnp.float32),
            pltpu.SemaphoreType.DMA,
        ],
    )
    def _k(len_hbm, tab_hbm, out_hbm, lens_v, zbuf, buf, semZ):
        c = lax.axis_index("c")
        s = lax.axis_index("s")
        wid = s * NC + c

        pltpu.sync_copy(len_hbm, lens_v.at[pl.ds(0, B)])
        lens16 = lens_v[...]

        def chunk_m(j):
            g = wid + NW * j
            t0 = (g % gpb) * C
            b = g // gpb
            len_b = lens16[0]
            for bb in range(1, B):
                len_b = jnp.where(b == bb, lens16[bb], len_b)
            return t0, jnp.clip(len_b - t0, 0, C), g * C

        zero16 = jnp.zeros((_LANES,), jnp.float32)

        def cnt(j, acc):
            _, m, _ = chunk_m(j)
            return acc + jnp.where(m == 0, 1, 0)

        npad = lax.fori_loop(0, my_chunks, cnt, 0)

        @pl.when(npad > 0)
        def _pads():
            def zrow(rp, carry):
                for g in range(D // _LANES):
                    zbuf[rp, pl.ds(g * _LANES, _LANES)] = zero16
                return carry

            lax.fori_loop(0, _ZROWS, zrow, 0)

            def fire(j, carry):
                _, m, o0 = chunk_m(j)

                @pl.when(m == 0)
                def _():
                    pltpu.make_async_copy(
                        zbuf, out_hbm.at[pl.ds(o0, _ZROWS)], semZ).start()
                    pltpu.make_async_copy(
                        zbuf.at[pl.ds(0, C - _ZROWS)],
                        out_hbm.at[pl.ds(o0 + _ZROWS, C - _ZROWS)],
                        semZ).start()
                return carry

            lax.fori_loop(0, my_chunks, fire, 0)

        def full(j, carry):
            t0, m, o0 = chunk_m(j)

            @pl.when(m == C)
            def _():
                pltpu.sync_copy(tab_hbm.at[pl.ds(t0, C)], buf)
                pltpu.sync_copy(buf, out_hbm.at[pl.ds(o0, C)])
            return carry

        lax.fori_loop(0, my_chunks, full, 0)

        def boundary(j, carry):
            t0, m, o0 = chunk_m(j)

            @pl.when((m > 0) & (m < C))
            def _():
                pltpu.sync_copy(tab_hbm.at[pl.ds(t0, C)], buf)

                def zrow(rp, carry2):
                    for g in range(D // _LANES):
                        buf[rp, pl.ds(g * _LANES, _LANES)] = zero16
                    return carry2

                lax.fori_loop(m, C, zrow, 0)
                pltpu.sync_copy(buf, out_hbm.at[pl.ds(o0, C)])
            return carry

        lax.fori_loop(0, my_chunks, boundary, 0)

        def drain(j, carry):
            pltpu.make_async_copy(
                zbuf, out_hbm.at[pl.ds(wid * C, _ZROWS)], semZ).wait()
            pltpu.make_async_copy(
                zbuf.at[pl.ds(0, C - _ZROWS)],
                out_hbm.at[pl.ds(wid * C, C - _ZROWS)], semZ).wait()
            return carry

        lax.fori_loop(0, npad, drain, 0)

    return _k(input_len, table2)


def _shift_body(a_ref, b_ref, o_ref):
    o_ref[...] = jnp.concatenate([a_ref[1:], b_ref[:1]], axis=0)


@jax.jit
def _shift_table(pos_table):
    V, D = pos_table.shape
    T = V - 1
    CB = 256
    return pl.pallas_call(
        _shift_body,
        grid=(T // CB,),
        in_specs=[
            pl.BlockSpec((CB, D), lambda r: (r, 0)),
            pl.BlockSpec((8, D), lambda r: ((r + 1) * (CB // 8), 0)),
        ],
        out_specs=pl.BlockSpec((CB, D), lambda r: (r, 0)),
        out_shape=jax.ShapeDtypeStruct((T, D), jnp.float32),
    )(pos_table, pos_table)


def kernel(input_len, max_len, pos_table):
    del max_len
    V, D = pos_table.shape
    T = V - 1
    B = input_len.shape[0]
    table2 = _shift_table(pos_table)
    out = _sc_expand(input_len, table2, B, T, D)
    return out.reshape(B, T, D)


# speedup vs baseline: 1.1626x; 1.0611x over previous
import functools

import jax
import jax.numpy as jnp
from jax import lax
from jax.experimental import pallas as pl
from jax.experimental.pallas import tpu as pltpu
from jax.experimental.pallas import tpu_sc as plsc

_LANES = 16
_CHUNK = 64
_ZROWS = 56


@functools.partial(jax.jit, static_argnums=(2, 3, 4))
def _sc_expand(input_len, table2, B, T, D):
    NC = 2
    NS = 16
    NW = NC * NS
    C = _CHUNK
    G = (B * T) // C
    gpb = T // C
    my_chunks = G // NW

    mesh = plsc.VectorSubcoreMesh(core_axis_name="c", subcore_axis_name="s")

    @functools.partial(
        pl.kernel,
        mesh=mesh,
        out_type=jax.ShapeDtypeStruct((B * T, D), jnp.float32),
        scratch_types=[
            pltpu.VMEM((_LANES,), jnp.int32),
            pltpu.VMEM((_ZROWS, D), jnp.float32),
            pltpu.VMEM((C, D), jnp.float32),
            pltpu.SemaphoreType.DMA,
        ],
    )
    def _k(len_hbm, tab_hbm, out_hbm, lens_v, zbuf, buf, semZ):
        c = lax.axis_index("c")
        s = lax.axis_index("s")
        wid = s * NC + c

        pltpu.sync_copy(len_hbm, lens_v.at[pl.ds(0, B)])
        lens16 = lens_v[...]

        def chunk_m(j):
            g = wid + NW * j
            t0 = (g % gpb) * C
            b = g // gpb
            len_b = lens16[0]
            for bb in range(1, B):
                len_b = jnp.where(b == bb, lens16[bb], len_b)
            return t0, jnp.clip(len_b - t0, 0, C), g * C

        zero16 = jnp.zeros((_LANES,), jnp.float32)

        def cnt(j, acc):
            _, m, _ = chunk_m(j)
            return acc + jnp.where(m == 0, 1, 0)

        npad = lax.fori_loop(0, my_chunks, cnt, 0)

        @pl.when(npad > 0)
        def _pads():
            def zrow(rp, carry):
                for g in range(D // _LANES):
                    zbuf[rp, pl.ds(g * _LANES, _LANES)] = zero16
                return carry

            lax.fori_loop(0, _ZROWS, zrow, 0)

            def fire(j, carry):
                _, m, o0 = chunk_m(j)

                @pl.when(m == 0)
                def _():
                    pltpu.make_async_copy(
                        zbuf, out_hbm.at[pl.ds(o0, _ZROWS)], semZ).start()
                    pltpu.make_async_copy(
                        zbuf.at[pl.ds(0, C - _ZROWS)],
                        out_hbm.at[pl.ds(o0 + _ZROWS, C - _ZROWS)],
                        semZ).start()
                return carry

            lax.fori_loop(0, my_chunks, fire, 0)

        def full(j, carry):
            t0, m, o0 = chunk_m(j)

            @pl.when(m == C)
            def _():
                pltpu.sync_copy(tab_hbm.at[pl.ds(t0, C)], buf)
                pltpu.sync_copy(buf, out_hbm.at[pl.ds(o0, C)])
            return carry

        lax.fori_loop(0, my_chunks, full, 0)

        def boundary(j, carry):
            t0, m, o0 = chunk_m(j)

            @pl.when((m > 0) & (m < C))
            def _():
                pltpu.sync_copy(tab_hbm.at[pl.ds(t0, C)], buf)

                def zrow(rp, carry2):
                    for g in range(D // _LANES):
                        buf[rp, pl.ds(g * _LANES, _LANES)] = zero16
                    return carry2

                lax.fori_loop(m, C, zrow, 0)
                pltpu.sync_copy(buf, out_hbm.at[pl.ds(o0, C)])
            return carry

        lax.fori_loop(0, my_chunks, boundary, 0)

        def drain(j, carry):
            pltpu.make_async_copy(
                zbuf, out_hbm.at[pl.ds(wid * C, _ZROWS)], semZ).wait()
            pltpu.make_async_copy(
                zbuf.at[pl.ds(0, C - _ZROWS)],
                out_hbm.at[pl.ds(wid * C, C - _ZROWS)], semZ).wait()
            return carry

        lax.fori_loop(0, npad, drain, 0)

    return _k(input_len, table2)


def _shift_body(a_ref, b_ref, o_ref):
    o_ref[...] = jnp.concatenate([a_ref[1:], b_ref[:1]], axis=0)


@jax.jit
def _shift_table(pos_table):
    V, D = pos_table.shape
    T = V - 1
    CB = 512
    return pl.pallas_call(
        _shift_body,
        grid=(T // CB,),
        in_specs=[
            pl.BlockSpec((CB, D), lambda r: (r, 0)),
            pl.BlockSpec((8, D), lambda r: ((r + 1) * (CB // 8), 0)),
        ],
        out_specs=pl.BlockSpec((CB, D), lambda r: (r, 0)),
        out_shape=jax.ShapeDtypeStruct((T, D), jnp.float32),
    )(pos_table, pos_table)


def kernel(input_len, max_len, pos_table):
    del max_len
    V, D = pos_table.shape
    T = V - 1
    B = input_len.shape[0]
    table2 = _shift_table(pos_table)
    out = _sc_expand(input_len, table2, B, T, D)
    return out.reshape(B, T, D)


# speedup vs baseline: 1.1778x; 1.0131x over previous
import functools

import jax
import jax.numpy as jnp
from jax import lax
from jax.experimental import pallas as pl
from jax.experimental.pallas import tpu as pltpu
from jax.experimental.pallas import tpu_sc as plsc

_LANES = 16
_CHUNK = 64
_ZROWS = 56


@functools.partial(jax.jit, static_argnums=(2, 3, 4))
def _sc_expand(input_len, table2, B, T, D):
    NC = 2
    NS = 16
    NW = NC * NS
    C = _CHUNK
    G = (B * T) // C
    gpb = T // C
    my_chunks = G // NW

    mesh = plsc.VectorSubcoreMesh(core_axis_name="c", subcore_axis_name="s")

    @functools.partial(
        pl.kernel,
        mesh=mesh,
        out_type=jax.ShapeDtypeStruct((B * T, D), jnp.float32),
        scratch_types=[
            pltpu.VMEM((_LANES,), jnp.int32),
            pltpu.VMEM((_ZROWS, D), jnp.float32),
            pltpu.VMEM((C, D), jnp.float32),
            pltpu.SemaphoreType.DMA,
        ],
    )
    def _k(len_hbm, tab_hbm, out_hbm, lens_v, zbuf, buf, semZ):
        c = lax.axis_index("c")
        s = lax.axis_index("s")
        wid = s * NC + c

        pltpu.sync_copy(len_hbm, lens_v.at[pl.ds(0, B)])
        lens16 = lens_v[...]

        def chunk_m(j):
            g = wid + NW * j
            t0 = (g % gpb) * C
            b = g // gpb
            len_b = lens16[0]
            for bb in range(1, B):
                len_b = jnp.where(b == bb, lens16[bb], len_b)
            return t0, jnp.clip(len_b - t0, 0, C), g * C

        zero16 = jnp.zeros((_LANES,), jnp.float32)

        def cnt(j, acc):
            _, m, _ = chunk_m(j)
            return acc + jnp.where(m == 0, 1, 0)

        npad = lax.fori_loop(0, my_chunks, cnt, 0)

        @pl.when(npad > 0)
        def _pads():
            def zrow(rp, carry):
                for g in range(D // _LANES):
                    zbuf[rp, pl.ds(g * _LANES, _LANES)] = zero16
                return carry

            lax.fori_loop(0, _ZROWS, zrow, 0)

            def fire(j, carry):
                _, m, o0 = chunk_m(j)

                @pl.when(m == 0)
                def _():
                    pltpu.make_async_copy(
                        zbuf, out_hbm.at[pl.ds(o0, _ZROWS)], semZ).start()
                    pltpu.make_async_copy(
                        zbuf.at[pl.ds(0, C - _ZROWS)],
                        out_hbm.at[pl.ds(o0 + _ZROWS, C - _ZROWS)],
                        semZ).start()
                return carry

            lax.fori_loop(0, my_chunks, fire, 0)

        def full(j, carry):
            t0, m, o0 = chunk_m(j)

            @pl.when(m == C)
            def _():
                pltpu.sync_copy(tab_hbm.at[pl.ds(t0, C)], buf)
                pltpu.sync_copy(buf, out_hbm.at[pl.ds(o0, C)])
            return carry

        lax.fori_loop(0, my_chunks, full, 0)

        def boundary(j, carry):
            t0, m, o0 = chunk_m(j)

            @pl.when((m > 0) & (m < C))
            def _():
                pltpu.sync_copy(tab_hbm.at[pl.ds(t0, C)], buf)

                def zrow(rp, carry2):
                    for g in range(D // _LANES):
                        buf[rp, pl.ds(g * _LANES, _LANES)] = zero16
                    return carry2

                lax.fori_loop(m, C, zrow, 0)
                pltpu.sync_copy(buf, out_hbm.at[pl.ds(o0, C)])
            return carry

        lax.fori_loop(0, my_chunks, boundary, 0)

        def drain(j, carry):
            pltpu.make_async_copy(
                zbuf, out_hbm.at[pl.ds(wid * C, _ZROWS)], semZ).wait()
            pltpu.make_async_copy(
                zbuf.at[pl.ds(0, C - _ZROWS)],
                out_hbm.at[pl.ds(wid * C, C - _ZROWS)], semZ).wait()
            return carry

        lax.fori_loop(0, npad, drain, 0)

    return _k(input_len, table2)


def _shift_body(a_ref, b_ref, o_ref):
    o_ref[...] = jnp.concatenate([a_ref[1:], b_ref[:1]], axis=0)


@jax.jit
def _shift_table(pos_table):
    V, D = pos_table.shape
    T = V - 1
    CB = 1024
    return pl.pallas_call(
        _shift_body,
        grid=(T // CB,),
        in_specs=[
            pl.BlockSpec((CB, D), lambda r: (r, 0)),
            pl.BlockSpec((8, D), lambda r: ((r + 1) * (CB // 8), 0)),
        ],
        out_specs=pl.BlockSpec((CB, D), lambda r: (r, 0)),
        out_shape=jax.ShapeDtypeStruct((T, D), jnp.float32),
    )(pos_table, pos_table)


def kernel(input_len, max_len, pos_table):
    del max_len
    V, D = pos_table.shape
    T = V - 1
    B = input_len.shape[0]
    table2 = _shift_table(pos_table)
    out = _sc_expand(input_len, table2, B, T, D)
    return out.reshape(B, T, D)


# speedup vs baseline: 1.1971x; 1.0164x over previous
import functools

import jax
import jax.numpy as jnp
from jax import lax
from jax.experimental import pallas as pl
from jax.experimental.pallas import tpu as pltpu
from jax.experimental.pallas import tpu_sc as plsc

_LANES = 16
_CHUNK = 64
_ZROWS = 56


@functools.partial(jax.jit, static_argnums=(2, 3, 4))
def _sc_expand(input_len, table2, B, T, D):
    NC = 2
    NS = 16
    NW = NC * NS
    C = _CHUNK
    G = (B * T) // C
    gpb = T // C
    my_chunks = G // NW

    mesh = plsc.VectorSubcoreMesh(core_axis_name="c", subcore_axis_name="s")

    @functools.partial(
        pl.kernel,
        mesh=mesh,
        out_type=jax.ShapeDtypeStruct((B * T, D), jnp.float32),
        scratch_types=[
            pltpu.VMEM((_LANES,), jnp.int32),
            pltpu.VMEM((_ZROWS, D), jnp.float32),
            pltpu.VMEM((C, D), jnp.float32),
            pltpu.SemaphoreType.DMA,
        ],
    )
    def _k(len_hbm, tab_hbm, out_hbm, lens_v, zbuf, buf, semZ):
        c = lax.axis_index("c")
        s = lax.axis_index("s")
        wid = s * NC + c

        pltpu.sync_copy(len_hbm, lens_v.at[pl.ds(0, B)])
        lens16 = lens_v[...]

        def chunk_m(j):
            g = wid + NW * j
            t0 = (g % gpb) * C
            b = g // gpb
            len_b = lens16[0]
            for bb in range(1, B):
                len_b = jnp.where(b == bb, lens16[bb], len_b)
            return t0, jnp.clip(len_b - t0, 0, C), g * C

        zero16 = jnp.zeros((_LANES,), jnp.float32)

        def cnt(j, acc):
            _, m, _ = chunk_m(j)
            return acc + jnp.where(m == 0, 1, 0)

        npad = lax.fori_loop(0, my_chunks, cnt, 0)

        @pl.when(npad > 0)
        def _pads():
            def zrow(rp, carry):
                for g in range(D // _LANES):
                    zbuf[rp, pl.ds(g * _LANES, _LANES)] = zero16
                return carry

            lax.fori_loop(0, _ZROWS, zrow, 0)

            def fire(j, carry):
                _, m, o0 = chunk_m(j)

                @pl.when(m == 0)
                def _():
                    pltpu.make_async_copy(
                        zbuf, out_hbm.at[pl.ds(o0, _ZROWS)], semZ).start()
                    pltpu.make_async_copy(
                        zbuf.at[pl.ds(0, C - _ZROWS)],
                        out_hbm.at[pl.ds(o0 + _ZROWS, C - _ZROWS)],
                        semZ).start()
                return carry

            lax.fori_loop(0, my_chunks, fire, 0)

        def full(j, carry):
            t0, m, o0 = chunk_m(j)

            @pl.when(m == C)
            def _():
                pltpu.sync_copy(tab_hbm.at[pl.ds(t0, C)], buf)
                pltpu.sync_copy(buf, out_hbm.at[pl.ds(o0, C)])
            return carry

        lax.fori_loop(0, my_chunks, full, 0)

        def boundary(j, carry):
            t0, m, o0 = chunk_m(j)

            @pl.when((m > 0) & (m < C))
            def _():
                pltpu.sync_copy(tab_hbm.at[pl.ds(t0, C)], buf)

                def zrow(rp, carry2):
                    for g in range(D // _LANES):
                        buf[rp, pl.ds(g * _LANES, _LANES)] = zero16
                    return carry2

                lax.fori_loop(m, C, zrow, 0)
                pltpu.sync_copy(buf, out_hbm.at[pl.ds(o0, C)])
            return carry

        lax.fori_loop(0, my_chunks, boundary, 0)

        def drain(j, carry):
            pltpu.make_async_copy(
                zbuf, out_hbm.at[pl.ds(wid * C, _ZROWS)], semZ).wait()
            pltpu.make_async_copy(
                zbuf.at[pl.ds(0, C - _ZROWS)],
                out_hbm.at[pl.ds(wid * C, C - _ZROWS)], semZ).wait()
            return carry

        lax.fori_loop(0, npad, drain, 0)

    return _k(input_len, table2)


def _shift_body(a_ref, b_ref, o_ref):
    o_ref[...] = jnp.concatenate([a_ref[1:], b_ref[:1]], axis=0)


@jax.jit
def _shift_table(pos_table):
    V, D = pos_table.shape
    T = V - 1
    CB = 2048
    return pl.pallas_call(
        _shift_body,
        grid=(T // CB,),
        in_specs=[
            pl.BlockSpec((CB, D), lambda r: (r, 0)),
            pl.BlockSpec((8, D), lambda r: ((r + 1) * (CB // 8), 0)),
        ],
        out_specs=pl.BlockSpec((CB, D), lambda r: (r, 0)),
        out_shape=jax.ShapeDtypeStruct((T, D), jnp.float32),
    )(pos_table, pos_table)


def kernel(input_len, max_len, pos_table):
    del max_len
    V, D = pos_table.shape
    T = V - 1
    B = input_len.shape[0]
    table2 = _shift_table(pos_table)
    out = _sc_expand(input_len, table2, B, T, D)
    return out.reshape(B, T, D)
